# baseline (speedup 1.0000x reference)
import functools

import jax
import jax.numpy as jnp
from jax import lax
from jax.experimental import pallas as pl
from jax.experimental.pallas import tpu as pltpu

_D = 128
_T = 20
_B = 32
_S = 5
_H = 50
_ROWS = 40
_NB = 1760 // _ROWS
_CBLK = 160 // _ROWS


def _elu(x):
    return jnp.where(x > 0, x, jnp.exp(x) - 1.0)


def _dense_body(t_raw_ref, node_ref, agg_ref,
                W_c1_ref, b_c1_ref, W_c2_ref, b_c2_ref,
                W_ae_ref, b_ae_ref, W_a1_ref, b_a1_ref, W_a2_ref,
                W_m1_ref, b_m1_ref, W_m2_ref, b_m2_ref,
                out_ref, c_scr, u_scr):
    g = pl.program_id(0)

    t = t_raw_ref[...]
    t = _elu(jnp.dot(t, W_c1_ref[...], preferred_element_type=jnp.float32)
             + b_c1_ref[...])
    t = jnp.tanh(jnp.dot(t, W_c2_ref[...], preferred_element_type=jnp.float32)
                 + b_c2_ref[...])

    node = node_ref[...]
    agg = agg_ref[...]
    W_ae = W_ae_ref[...]
    a = jnp.tanh(jnp.dot(node, W_ae[:_D], preferred_element_type=jnp.float32)
                 + jnp.dot(agg, W_ae[_D:], preferred_element_type=jnp.float32)
                 + b_ae_ref[...])
    h = _elu(jnp.dot(a, W_a1_ref[...], preferred_element_type=jnp.float32)
             + b_a1_ref[...])
    w2 = W_a2_ref[...].reshape(1, _D)
    h3 = h.reshape(_ROWS, _T, _D)
    logits = jnp.sum(h3 * w2[None], axis=-1)
    m = jnp.max(logits, axis=1, keepdims=True)
    e = jnp.exp(logits - m)
    w = e / jnp.sum(e, axis=1, keepdims=True)
    a3 = a.reshape(_ROWS, _T, _D)
    anchor = jnp.sum(a3 * w[:, :, None], axis=1)

    W_m1 = W_m1_ref[...]
    y = _elu(jnp.dot(t, W_m1[:_D], preferred_element_type=jnp.float32)
             + jnp.dot(anchor, W_m1[_D:], preferred_element_type=jnp.float32)
             + b_m1_ref[...])
    y = _elu(jnp.dot(y, W_m2_ref[...], preferred_element_type=jnp.float32)
             + b_m2_ref[...])

    @pl.when(g == 0)
    def _():
        u_scr[...] = jnp.zeros_like(u_scr)

    @pl.when(g < _CBLK)
    def _():
        c_scr[pl.ds(g * _ROWS, _ROWS), :] = y

    @pl.when(g >= _CBLK)
    def _():
        rows = (g - _CBLK) * _ROWS + lax.broadcasted_iota(jnp.int32, (_B, _ROWS), 1)
        sel = (rows // _H == lax.broadcasted_iota(jnp.int32, (_B, _ROWS), 0))
        u_scr[...] += jnp.dot(sel.astype(jnp.float32), y,
                              preferred_element_type=jnp.float32) * (1.0 / _H)

    @pl.when(g == _NB - 1)
    def _():
        u = u_scr[...]
        c3 = c_scr[...].reshape(_B, _S, _D)
        out_ref[...] = jnp.sum(c3 * u[:, None, :], axis=-1)


def _dense_call(t_raw, node_e, agg, W_c1, b_c1, W_c2, b_c2,
                W_ae, b_ae, W_a1, b_a1, W_a2, W_m1, b_m1, W_m2, b_m2):
    full2 = lambda arr: pl.BlockSpec(arr.shape, lambda g: (0,) * arr.ndim)
    return pl.pallas_call(
        _dense_body,
        grid=(_NB,),
        in_specs=[
            pl.BlockSpec((_ROWS, 768), lambda g: (g, 0)),
            pl.BlockSpec((_ROWS * _T, _D), lambda g: (g, 0)),
            pl.BlockSpec((_ROWS * _T, _D), lambda g: (g, 0)),
            full2(W_c1), full2(b_c1), full2(W_c2), full2(b_c2),
            full2(W_ae), full2(b_ae), full2(W_a1), full2(b_a1), full2(W_a2),
            full2(W_m1), full2(b_m1), full2(W_m2), full2(b_m2),
        ],
        out_specs=pl.BlockSpec((_B, _S), lambda g: (0, 0)),
        out_shape=jax.ShapeDtypeStruct((_B, _S), jnp.float32),
        scratch_shapes=[
            pltpu.VMEM((_CBLK * _ROWS, _D), jnp.float32),
            pltpu.VMEM((_B, _D), jnp.float32),
        ],
    )(t_raw, node_e, agg, W_c1, b_c1, W_c2, b_c2,
      W_ae, b_ae, W_a1, b_a1, W_a2, W_m1, b_m1, W_m2, b_m2)


def kernel(cand_news, clicked_news, cand_anchor_graph1, clicked_anchor_graph2,
           entity_adj, relation_adj, news_title_embedding, entity_embedding,
           relation_embedding, W_c1, b_c1, W_c2, b_c2, W_m1, b_m1, W_m2, b_m2,
           W_ae, b_ae, W_a1, b_a1, W_a2, b_a2):
    del b_a2

    news_flat = jnp.concatenate([cand_news.reshape(-1),
                                 clicked_news.reshape(-1)])
    nodes_flat = jnp.concatenate([cand_anchor_graph1.reshape(-1),
                                  clicked_anchor_graph2.reshape(-1)])

    t_raw = jnp.take(news_title_embedding, news_flat, axis=0)
    node_e = jnp.take(entity_embedding, nodes_flat, axis=0)
    nb_e = jnp.take(entity_adj, nodes_flat, axis=0)
    nb_r = jnp.take(relation_adj, nodes_flat, axis=0)
    agg = (jnp.take(entity_embedding, nb_e.reshape(-1), axis=0)
           .reshape(-1, 10, _D).sum(axis=1)
           + jnp.take(relation_embedding, nb_r.reshape(-1), axis=0)
           .reshape(-1, 10, _D).sum(axis=1))

    return _dense_call(t_raw, node_e, agg, W_c1, b_c1, W_c2, b_c2,
                       W_ae, b_ae, W_a1, b_a1, W_a2, W_m1, b_m1, W_m2, b_m2)


# speedup vs baseline: 2.4599x; 2.4599x over previous
import functools

import jax
import jax.numpy as jnp
from jax import lax
from jax.experimental import pallas as pl
from jax.experimental.pallas import tpu as pltpu
from jax.experimental.pallas import tpu_sc as plsc

_D = 128
_T = 20
_B = 32
_S = 5
_H = 50
_ROWS = 40
_NB = 1760 // _ROWS
_CBLK = 160 // _ROWS

_K = 10
_NW = 32
_N_NODES = 35200
_WPN = 1104
_N_PAD = _NW * _WPN
_C = 48
_SUB = _WPN // _C


def _elu(x):
    return jnp.where(x > 0, x, jnp.exp(x) - 1.0)


def _sc_body(nodes_hbm, flate_hbm, flatr_hbm, ent_hbm, rel_hbm,
             node_out, agg_out,
             idx_v, flate_v, flatr_v, noderows_v, nb_v, agg_v,
             sem0, sem1, sem2):
    wid = lax.axis_index("s") * 2 + lax.axis_index("c")

    def sub(s, carry):
        base = wid * _WPN + s * _C
        pltpu.sync_copy(nodes_hbm.at[pl.ds(base, _C)], idx_v)
        cp_n = pltpu.async_copy(ent_hbm.at[idx_v], noderows_v, sem2)
        pltpu.sync_copy(flate_hbm.at[pl.ds(base * _K, _C * _K)], flate_v)
        pltpu.sync_copy(flatr_hbm.at[pl.ds(base * _K, _C * _K)], flatr_v)

        def red(nb_ref, out_ref, accumulate):
            def body(c2, carry2):
                for col in range(_D // 16):
                    sl = pl.ds(col * 16, 16)
                    acc = nb_ref[c2 * _K, sl]
                    for k in range(1, _K):
                        acc = acc + nb_ref[c2 * _K + k, sl]
                    if accumulate:
                        out_ref[c2, sl] += acc
                    else:
                        out_ref[c2, sl] = acc
                return carry2
            lax.fori_loop(0, _C, body, 0)

        pltpu.async_copy(ent_hbm.at[flate_v], nb_v, sem0).wait()
        red(nb_v, agg_v, False)
        pltpu.async_copy(rel_hbm.at[flatr_v], nb_v, sem1).wait()
        red(nb_v, agg_v, True)
        cp_n.wait()
        pltpu.sync_copy(noderows_v, node_out.at[pl.ds(base, _C)])
        pltpu.sync_copy(agg_v, agg_out.at[pl.ds(base, _C)])
        return carry

    lax.fori_loop(0, _SUB, sub, 0)


def _sc_gather(nodes_pad, flate, flatr, ent_emb, rel_emb):
    mesh = plsc.VectorSubcoreMesh(core_axis_name="c", subcore_axis_name="s")
    f32, i32 = jnp.float32, jnp.int32
    return pl.kernel(
        _sc_body,
        out_type=(jax.ShapeDtypeStruct((_N_PAD, _D), f32),
                  jax.ShapeDtypeStruct((_N_PAD, _D), f32)),
        mesh=mesh,
        scratch_types=[
            pltpu.VMEM((_C,), i32),
            pltpu.VMEM((_C * _K,), i32),
            pltpu.VMEM((_C * _K,), i32),
            pltpu.VMEM((_C, _D), f32),
            pltpu.VMEM((_C * _K, _D), f32),
            pltpu.VMEM((_C, _D), f32),
            pltpu.SemaphoreType.DMA,
            pltpu.SemaphoreType.DMA,
            pltpu.SemaphoreType.DMA,
        ],
    )(nodes_pad, flate, flatr, ent_emb, rel_emb)


def _dense_body(t_raw_ref, node_ref, agg_ref,
                W_c1_ref, b_c1_ref, W_c2_ref, b_c2_ref,
                W_ae_ref, b_ae_ref, W_a1_ref, b_a1_ref, W_a2_ref,
                W_m1_ref, b_m1_ref, W_m2_ref, b_m2_ref,
                out_ref, c_scr, u_scr):
    g = pl.program_id(0)

    t = t_raw_ref[...]
    t = _elu(jnp.dot(t, W_c1_ref[...], preferred_element_type=jnp.float32)
             + b_c1_ref[...])
    t = jnp.tanh(jnp.dot(t, W_c2_ref[...], preferred_element_type=jnp.float32)
                 + b_c2_ref[...])

    node = node_ref[...]
    agg = agg_ref[...]
    W_ae = W_ae_ref[...]
    a = jnp.tanh(jnp.dot(node, W_ae[:_D], preferred_element_type=jnp.float32)
                 + jnp.dot(agg, W_ae[_D:], preferred_element_type=jnp.float32)
                 + b_ae_ref[...])
    h = _elu(jnp.dot(a, W_a1_ref[...], preferred_element_type=jnp.float32)
             + b_a1_ref[...])
    w2 = W_a2_ref[...].reshape(1, _D)
    h3 = h.reshape(_ROWS, _T, _D)
    logits = jnp.sum(h3 * w2[None], axis=-1)
    m = jnp.max(logits, axis=1, keepdims=True)
    e = jnp.exp(logits - m)
    w = e / jnp.sum(e, axis=1, keepdims=True)
    a3 = a.reshape(_ROWS, _T, _D)
    anchor = jnp.sum(a3 * w[:, :, None], axis=1)

    W_m1 = W_m1_ref[...]
    y = _elu(jnp.dot(t, W_m1[:_D], preferred_element_type=jnp.float32)
             + jnp.dot(anchor, W_m1[_D:], preferred_element_type=jnp.float32)
             + b_m1_ref[...])
    y = _elu(jnp.dot(y, W_m2_ref[...], preferred_element_type=jnp.float32)
             + b_m2_ref[...])

    @pl.when(g == 0)
    def _():
        u_scr[...] = jnp.zeros_like(u_scr)

    @pl.when(g < _CBLK)
    def _():
        c_scr[pl.ds(g * _ROWS, _ROWS), :] = y

    @pl.when(g >= _CBLK)
    def _():
        rows = (g - _CBLK) * _ROWS + lax.broadcasted_iota(jnp.int32, (_B, _ROWS), 1)
        sel = (rows // _H == lax.broadcasted_iota(jnp.int32, (_B, _ROWS), 0))
        u_scr[...] += jnp.dot(sel.astype(jnp.float32), y,
                              preferred_element_type=jnp.float32) * (1.0 / _H)

    @pl.when(g == _NB - 1)
    def _():
        u = u_scr[...]
        c3 = c_scr[...].reshape(_B, _S, _D)
        out_ref[...] = jnp.sum(c3 * u[:, None, :], axis=-1)


def _dense_call(t_raw, node_e, agg, W_c1, b_c1, W_c2, b_c2,
                W_ae, b_ae, W_a1, b_a1, W_a2, W_m1, b_m1, W_m2, b_m2):
    full2 = lambda arr: pl.BlockSpec(arr.shape, lambda g: (0,) * arr.ndim)
    return pl.pallas_call(
        _dense_body,
        grid=(_NB,),
        in_specs=[
            pl.BlockSpec((_ROWS, 768), lambda g: (g, 0)),
            pl.BlockSpec((_ROWS * _T, _D), lambda g: (g, 0)),
            pl.BlockSpec((_ROWS * _T, _D), lambda g: (g, 0)),
            full2(W_c1), full2(b_c1), full2(W_c2), full2(b_c2),
            full2(W_ae), full2(b_ae), full2(W_a1), full2(b_a1), full2(W_a2),
            full2(W_m1), full2(b_m1), full2(W_m2), full2(b_m2),
        ],
        out_specs=pl.BlockSpec((_B, _S), lambda g: (0, 0)),
        out_shape=jax.ShapeDtypeStruct((_B, _S), jnp.float32),
        scratch_shapes=[
            pltpu.VMEM((_CBLK * _ROWS, _D), jnp.float32),
            pltpu.VMEM((_B, _D), jnp.float32),
        ],
    )(t_raw, node_e, agg, W_c1, b_c1, W_c2, b_c2,
      W_ae, b_ae, W_a1, b_a1, W_a2, W_m1, b_m1, W_m2, b_m2)


def kernel(cand_news, clicked_news, cand_anchor_graph1, clicked_anchor_graph2,
           entity_adj, relation_adj, news_title_embedding, entity_embedding,
           relation_embedding, W_c1, b_c1, W_c2, b_c2, W_m1, b_m1, W_m2, b_m2,
           W_ae, b_ae, W_a1, b_a1, W_a2, b_a2):
    del b_a2

    news_flat = jnp.concatenate([cand_news.reshape(-1),
                                 clicked_news.reshape(-1)])
    nodes_flat = jnp.concatenate([cand_anchor_graph1.reshape(-1),
                                  clicked_anchor_graph2.reshape(-1)])

    t_raw = jnp.take(news_title_embedding, news_flat, axis=0)
    nodes_pad = jnp.pad(nodes_flat, (0, _N_PAD - _N_NODES))
    flate = jnp.take(entity_adj, nodes_pad, axis=0).reshape(-1)
    flatr = jnp.take(relation_adj, nodes_pad, axis=0).reshape(-1)
    node_e, agg = _sc_gather(nodes_pad, flate, flatr,
                             entity_embedding, relation_embedding)

    return _dense_call(t_raw, node_e, agg, W_c1, b_c1, W_c2, b_c2,
                       W_ae, b_ae, W_a1, b_a1, W_a2, W_m1, b_m1, W_m2, b_m2)
